# baseline (speedup 1.0000x reference)
import jax
import jax.numpy as jnp
from jax.experimental import pallas as pl


_BLKN = 8192


def kernel(psi):
    rows, cols = psi.shape
    nb = rows // _BLKN

    def body(x_ref, o_ref):
        i = pl.program_id(0)

        @pl.when(i < nb)
        def _copy():
            o_ref[...] = x_ref[...]

        @pl.when(i >= nb)
        def _zero():
            o_ref[...] = jnp.zeros_like(o_ref)

    return pl.pallas_call(
        body,
        grid=(2 * nb,),
        in_specs=[pl.BlockSpec((_BLKN, cols), lambda i: (jnp.minimum(i, nb - 1), 0))],
        out_specs=pl.BlockSpec((_BLKN, cols), lambda i: (i, 0)),
        out_shape=jax.ShapeDtypeStruct((2 * rows, cols), psi.dtype),
    )(psi)
